# baseline (speedup 1.0000x reference)
import functools

import jax
import jax.numpy as jnp
from jax import lax
from jax.experimental import pallas as pl
from jax.experimental.pallas import tpu as pltpu
from jax.experimental.pallas import tpu_sc as plsc

N = 10000
F = 128
FW = 144
NINC = 320000
NC = 2
NS = 16
NW = NC * NS
CH = 64
NCH = 160
NPAD = NW * NCH * CH
RND = 80
SLOW = 1
R_SLOW = 1
R_FAST = 3
FAST_BASE = NS * R_SLOW * RND
ACC_ROWS = 10112
TROWS = ACC_ROWS // NS
PAD = N

_mesh = plsc.VectorSubcoreMesh(core_axis_name="c", subcore_axis_name="s",
                               num_cores=NC, num_subcores=NS)


@functools.partial(
    pl.kernel,
    out_type=jax.ShapeDtypeStruct((NC, ACC_ROWS, FW), jnp.float32),
    mesh=_mesh,
    scratch_types=[
        pltpu.VMEM((RND, CH), jnp.int32),
        pltpu.VMEM((RND, CH), jnp.int32),
        pltpu.VMEM((CH, FW), jnp.float32),
        pltpu.VMEM((CH, FW), jnp.float32),
        pltpu.VMEM_SHARED((ACC_ROWS, FW), jnp.float32),
        pltpu.SemaphoreType.DMA,
        pltpu.SemaphoreType.DMA,
    ],
    compiler_params=pltpu.CompilerParams(use_tc_tiling_on_sc=False),
)
def _sc_seg_sum(tbl, gidx, sidx, zblk, out, gi, si, rows_a, rows_b, acc,
                sem_a, sem_b):
    c = lax.axis_index("c")
    s = lax.axis_index("s")
    nrounds = jnp.where(c == SLOW, R_SLOW, R_FAST)
    cbase = jnp.where(c == SLOW, 0, FAST_BASE)
    pltpu.sync_copy(zblk, rows_a)
    for j in range(TROWS // CH):
        pltpu.sync_copy(rows_a, acc.at[pl.ds(s * TROWS + j * CH, CH)])
    pltpu.sync_copy(rows_a.at[pl.ds(0, TROWS % CH)],
                    acc.at[pl.ds(s * TROWS + (TROWS // CH) * CH, TROWS % CH)])
    plsc.subcore_barrier()

    def round_body(r, carry):
        row0 = cbase + (s * nrounds + r) * RND
        pltpu.sync_copy(gidx.at[pl.ds(row0, RND)], gi)
        pltpu.sync_copy(sidx.at[pl.ds(row0, RND)], si)
        pltpu.async_copy(tbl.at[gi.at[0]], rows_a, sem_a)

        def body(j, carry2):
            c0 = 2 * j
            pltpu.async_copy(tbl.at[gi.at[c0 + 1]], rows_b, sem_b)
            pltpu.make_async_copy(tbl.at[gi.at[c0]], rows_a, sem_a).wait()
            pltpu.sync_copy(rows_a, acc.at[si.at[c0]], add=True)
            pltpu.async_copy(tbl.at[gi.at[c0 + 2]], rows_a, sem_a)
            pltpu.make_async_copy(tbl.at[gi.at[c0 + 1]], rows_b, sem_b).wait()
            pltpu.sync_copy(rows_b, acc.at[si.at[c0 + 1]], add=True)
            return carry2

        lax.fori_loop(0, RND // 2 - 1, body, 0)
        pltpu.async_copy(tbl.at[gi.at[RND - 1]], rows_b, sem_b)
        pltpu.make_async_copy(tbl.at[gi.at[RND - 2]], rows_a, sem_a).wait()
        pltpu.sync_copy(rows_a, acc.at[si.at[RND - 2]], add=True)
        pltpu.make_async_copy(tbl.at[gi.at[RND - 1]], rows_b, sem_b).wait()
        pltpu.sync_copy(rows_b, acc.at[si.at[RND - 1]], add=True)
        return carry

    lax.fori_loop(0, nrounds, round_body, 0)
    plsc.subcore_barrier()
    for j in range(TROWS // CH):
        r0 = s * TROWS + j * CH
        pltpu.sync_copy(acc.at[pl.ds(r0, CH)], rows_a)
        pltpu.sync_copy(rows_a, out.at[c, pl.ds(r0, CH)])
    r0 = s * TROWS + (TROWS // CH) * CH
    pltpu.sync_copy(acc.at[pl.ds(r0, TROWS % CH)], rows_a.at[pl.ds(0, TROWS % CH)])
    pltpu.sync_copy(rows_a.at[pl.ds(0, TROWS % CH)], out.at[c, pl.ds(r0, TROWS % CH)])


R = 1000


def _ln_leaky(h, g, b):
    mu = jnp.mean(h, axis=-1, keepdims=True)
    var = jnp.mean((h - mu) ** 2, axis=-1, keepdims=True)
    h = (h - mu) / jnp.sqrt(var + 1e-5) * g + b
    return jnp.where(h >= 0, h, 0.2 * h)


def _pack(feats, extra):
    return jnp.concatenate(
        [feats, extra, jnp.zeros((feats.shape[0], FW - F - 1), jnp.float32)],
        axis=1)


def _tc_pre_body(x_ref, w1_ref, wb_ref, bb_ref, gb_ref, beb_ref,
                 t1_ref, zb_ref):
    xb = x_ref[...]
    xw1 = jnp.dot(xb, w1_ref[...], preferred_element_type=jnp.float32)
    t1_ref[...] = _pack(xw1, jnp.ones((xw1.shape[0], 1), jnp.float32))
    h = jnp.dot(xb, wb_ref[...], preferred_element_type=jnp.float32) + bb_ref[...]
    zb_ref[...] = _ln_leaky(h, gb_ref[...], beb_ref[...])


def _tc_scale_body(sp_ref, hw_ref, t_ref):
    q = sp_ref[0] + sp_ref[1]
    bcnt = q[:, F:F + 1]
    binv = jnp.where(bcnt > 0, 1.0 / jnp.where(bcnt > 0, bcnt, 1.0), 0.0)
    t_ref[...] = _pack(binv * q[:, :F], hw_ref[...])


def _tc_z1_body(sp_ref, b1_ref, g1_ref, be1_ref, w2_ref, z1_ref, t3_ref):
    q = sp_ref[0] + sp_ref[1]
    d = q[:, F:F + 1]
    dinv = jnp.where(d > 0, 1.0 / jnp.where(d > 0, d, 1.0), 0.0)
    h = dinv * q[:, :F] + b1_ref[...]
    z1 = _ln_leaky(h, g1_ref[...], be1_ref[...])
    z1_ref[...] = z1
    xw2 = jnp.dot(z1, w2_ref[...], preferred_element_type=jnp.float32)
    t3_ref[...] = _pack(xw2, jnp.ones((xw2.shape[0], 1), jnp.float32))


def _tc_tail_body(sp_ref, b2_ref, g2_ref, be2_ref, z1_ref, zb_ref,
                  wg1_ref, bg1_ref, wg2_ref, bg2_ref, wc_ref, bc_ref, out_ref):
    q = sp_ref[0] + sp_ref[1]
    d = q[:, F:F + 1]
    dinv = jnp.where(d > 0, 1.0 / jnp.where(d > 0, d, 1.0), 0.0)
    h = dinv * q[:, :F] + b2_ref[...]
    z2 = _ln_leaky(h, g2_ref[...], be2_ref[...])
    zsp = z1_ref[...] + z2
    zb = zb_ref[...]
    comb = jnp.concatenate([zsp, zb], axis=1)
    se = jax.nn.relu(jnp.dot(comb, wg1_ref[...], preferred_element_type=jnp.float32)
                     + bg1_ref[...])
    gate = jax.nn.sigmoid(jnp.dot(se, wg2_ref[...], preferred_element_type=jnp.float32)
                          + bg2_ref[...])
    fused = gate * zsp + (1.0 - gate) * zb
    out_ref[...] = jnp.dot(fused, wc_ref[...], preferred_element_type=jnp.float32) + bc_ref[...]


def _row_spec(width):
    return pl.BlockSpec((R, width), lambda i: (i, 0))


def _part_spec(width):
    return pl.BlockSpec((NC, R, width), lambda i: (0, i, 0))


def _full_spec(shape):
    nd = len(shape)
    return pl.BlockSpec(shape, lambda i: (0,) * nd)


def kernel(x, edge_index, edge_weight, W1, b1, g1, be1, W2, b2, g2, be2,
           Wb, bb, gb, beb, Wg1, bg1, Wg2, bg2, Wc, bc):
    node = edge_index[0]
    he = edge_index[1]
    padlen = NPAD - NINC
    node_g = jnp.pad(node, (0, padlen)).reshape(NW * NCH, CH)
    he_g = jnp.pad(he, (0, padlen)).reshape(NW * NCH, CH)
    node_s = jnp.pad(node, (0, padlen), constant_values=PAD).reshape(NW * NCH, CH)
    he_s = jnp.pad(he, (0, padlen), constant_values=PAD).reshape(NW * NCH, CH)
    zblk = jnp.zeros((CH, FW), jnp.float32)
    hw_col = edge_weight.reshape(N, 1)

    grid = (N // R,)
    t1, zb = pl.pallas_call(
        _tc_pre_body,
        grid=grid,
        in_specs=[_row_spec(F), _full_spec((F, F)), _full_spec((F, F)),
                  _full_spec((F,)), _full_spec((F,)), _full_spec((F,))],
        out_specs=[_row_spec(FW), _row_spec(F)],
        out_shape=[jax.ShapeDtypeStruct((N, FW), jnp.float32),
                   jax.ShapeDtypeStruct((N, F), jnp.float32)],
    )(x, W1, Wb, bb, gb, beb)

    def sc_pass(tbl, gidx, sidx):
        return _sc_seg_sum(tbl, gidx, sidx, zblk)

    def tc_scale(spart):
        return pl.pallas_call(
            _tc_scale_body,
            grid=grid,
            in_specs=[_part_spec(FW), _row_spec(1)],
            out_specs=_row_spec(FW),
            out_shape=jax.ShapeDtypeStruct((N, FW), jnp.float32),
        )(spart, hw_col)

    s1 = sc_pass(t1, node_g, he_s)
    t2 = tc_scale(s1)
    s2 = sc_pass(t2, he_g, node_s)
    z1, t3 = pl.pallas_call(
        _tc_z1_body,
        grid=grid,
        in_specs=[_part_spec(FW), _full_spec((F,)), _full_spec((F,)),
                  _full_spec((F,)), _full_spec((F, F))],
        out_specs=[_row_spec(F), _row_spec(FW)],
        out_shape=[jax.ShapeDtypeStruct((N, F), jnp.float32),
                   jax.ShapeDtypeStruct((N, FW), jnp.float32)],
    )(s2, b1, g1, be1, W2)

    s3 = sc_pass(t3, node_g, he_s)
    t4 = tc_scale(s3)
    s4 = sc_pass(t4, he_g, node_s)

    out = pl.pallas_call(
        _tc_tail_body,
        grid=grid,
        in_specs=[_part_spec(FW), _full_spec((F,)), _full_spec((F,)),
                  _full_spec((F,)), _row_spec(F), _row_spec(F),
                  _full_spec((2 * F, F // 2)), _full_spec((F // 2,)),
                  _full_spec((F // 2, F)), _full_spec((F,)),
                  _full_spec((F, 1)), _full_spec((1,))],
        out_specs=pl.BlockSpec((R, 1), lambda i: (i, 0)),
        out_shape=jax.ShapeDtypeStruct((N, 1), jnp.float32),
    )(s4, b2, g2, be2, z1, zb, Wg1, bg1, Wg2, bg2, Wc, bc)
    return out


# speedup vs baseline: 1.9134x; 1.9134x over previous
import functools

import jax
import jax.numpy as jnp
from jax import lax
from jax.experimental import pallas as pl
from jax.experimental.pallas import tpu as pltpu
from jax.experimental.pallas import tpu_sc as plsc

N = 10000
F = 128
FH = 80
HF = F // 2
NINC = 320000
NC = 2
NS = 16
NW = NC * NS
CH = 128
RND = 40
NRND = 2
NCHW = RND * NRND
NPAD = NW * NCHW * CH
ACC_ROWS = 10016
TROWS = ACC_ROWS // NS
PAD = N

_mesh = plsc.VectorSubcoreMesh(core_axis_name="c", subcore_axis_name="s",
                               num_cores=NC, num_subcores=NS)


@functools.partial(
    pl.kernel,
    out_type=jax.ShapeDtypeStruct((NC, 2, ACC_ROWS, FH), jnp.float32),
    mesh=_mesh,
    scratch_types=[
        pltpu.VMEM((RND, CH), jnp.int32),
        pltpu.VMEM((RND, CH), jnp.int32),
        pltpu.VMEM((CH, FH), jnp.float32),
        pltpu.VMEM((CH, FH), jnp.float32),
        pltpu.VMEM_SHARED((ACC_ROWS, FH), jnp.float32),
        pltpu.VMEM_SHARED((ACC_ROWS, FH), jnp.float32),
        pltpu.SemaphoreType.DMA,
        pltpu.SemaphoreType.DMA,
    ],
    compiler_params=pltpu.CompilerParams(use_tc_tiling_on_sc=False),
)
def _sc_seg_sum(tbl, gidx, sidx, zblk, out, gi, si, rows_a, rows_b,
                tsp, acc, sem_a, sem_b):
    c = lax.axis_index("c")
    s = lax.axis_index("s")
    w = c * NS + s
    r0 = s * TROWS
    nfull = TROWS // CH
    rem = TROWS % CH

    for h in range(2):
        for j in range(nfull):
            pltpu.sync_copy(tbl.at[h, pl.ds(r0 + j * CH, CH)], rows_a)
            pltpu.sync_copy(rows_a, tsp.at[pl.ds(r0 + j * CH, CH)])
        pltpu.sync_copy(tbl.at[h, pl.ds(r0 + nfull * CH, rem)],
                        rows_a.at[pl.ds(0, rem)])
        pltpu.sync_copy(rows_a.at[pl.ds(0, rem)],
                        tsp.at[pl.ds(r0 + nfull * CH, rem)])
        pltpu.sync_copy(zblk, rows_a)
        for j in range(nfull):
            pltpu.sync_copy(rows_a, acc.at[pl.ds(r0 + j * CH, CH)])
        pltpu.sync_copy(rows_a.at[pl.ds(0, rem)],
                        acc.at[pl.ds(r0 + nfull * CH, rem)])
        plsc.subcore_barrier()

        def round_body(r, carry):
            row0 = (w * NRND + r) * RND
            pltpu.sync_copy(gidx.at[pl.ds(row0, RND)], gi)
            pltpu.sync_copy(sidx.at[pl.ds(row0, RND)], si)
            pltpu.async_copy(tsp.at[gi.at[0]], rows_a, sem_a)

            def body(j, carry2):
                c0 = 2 * j
                pltpu.async_copy(tsp.at[gi.at[c0 + 1]], rows_b, sem_b)
                pltpu.make_async_copy(tsp.at[gi.at[c0]], rows_a, sem_a).wait()
                pltpu.sync_copy(rows_a, acc.at[si.at[c0]], add=True)
                pltpu.async_copy(tsp.at[gi.at[c0 + 2]], rows_a, sem_a)
                pltpu.make_async_copy(tsp.at[gi.at[c0 + 1]], rows_b, sem_b).wait()
                pltpu.sync_copy(rows_b, acc.at[si.at[c0 + 1]], add=True)
                return carry2

            lax.fori_loop(0, RND // 2 - 1, body, 0)
            pltpu.async_copy(tsp.at[gi.at[RND - 1]], rows_b, sem_b)
            pltpu.make_async_copy(tsp.at[gi.at[RND - 2]], rows_a, sem_a).wait()
            pltpu.sync_copy(rows_a, acc.at[si.at[RND - 2]], add=True)
            pltpu.make_async_copy(tsp.at[gi.at[RND - 1]], rows_b, sem_b).wait()
            pltpu.sync_copy(rows_b, acc.at[si.at[RND - 1]], add=True)
            return carry

        lax.fori_loop(0, NRND, round_body, 0)
        plsc.subcore_barrier()
        for j in range(nfull):
            pltpu.sync_copy(acc.at[pl.ds(r0 + j * CH, CH)], rows_a)
            pltpu.sync_copy(rows_a, out.at[c, h, pl.ds(r0 + j * CH, CH)])
        pltpu.sync_copy(acc.at[pl.ds(r0 + nfull * CH, rem)], rows_a.at[pl.ds(0, rem)])
        pltpu.sync_copy(rows_a.at[pl.ds(0, rem)], out.at[c, h, pl.ds(r0 + nfull * CH, rem)])


R = 1000


def _ln_leaky(h, g, b):
    mu = jnp.mean(h, axis=-1, keepdims=True)
    var = jnp.mean((h - mu) ** 2, axis=-1, keepdims=True)
    h = (h - mu) / jnp.sqrt(var + 1e-5) * g + b
    return jnp.where(h >= 0, h, 0.2 * h)


def _pack_halves(feats, extra):
    z = jnp.zeros((feats.shape[0], FH - HF - 1), jnp.float32)
    ta = jnp.concatenate([feats[:, :HF], extra, z], axis=1)
    tb = jnp.concatenate([feats[:, HF:], jnp.zeros_like(extra), z], axis=1)
    return jnp.stack([ta, tb])


def _unpack_sums(sp):
    qa = sp[0, 0] + sp[1, 0]
    qb = sp[0, 1] + sp[1, 1]
    return jnp.concatenate([qa[:, :HF], qb[:, :HF]], axis=1), qa[:, HF:HF + 1]


def _tc_pre_body(x_ref, w1_ref, wb_ref, bb_ref, gb_ref, beb_ref,
                 t1_ref, zb_ref):
    xb = x_ref[...]
    xw1 = jnp.dot(xb, w1_ref[...], preferred_element_type=jnp.float32)
    t1_ref[...] = _pack_halves(xw1, jnp.ones((xw1.shape[0], 1), jnp.float32))
    h = jnp.dot(xb, wb_ref[...], preferred_element_type=jnp.float32) + bb_ref[...]
    zb_ref[...] = _ln_leaky(h, gb_ref[...], beb_ref[...])


def _tc_scale_body(sp_ref, hw_ref, t_ref):
    q, bcnt = _unpack_sums(sp_ref[...])
    binv = jnp.where(bcnt > 0, 1.0 / jnp.where(bcnt > 0, bcnt, 1.0), 0.0)
    t_ref[...] = _pack_halves(binv * q, hw_ref[...])


def _tc_z1_body(sp_ref, b1_ref, g1_ref, be1_ref, w2_ref, z1_ref, t3_ref):
    q, d = _unpack_sums(sp_ref[...])
    dinv = jnp.where(d > 0, 1.0 / jnp.where(d > 0, d, 1.0), 0.0)
    h = dinv * q + b1_ref[...]
    z1 = _ln_leaky(h, g1_ref[...], be1_ref[...])
    z1_ref[...] = z1
    xw2 = jnp.dot(z1, w2_ref[...], preferred_element_type=jnp.float32)
    t3_ref[...] = _pack_halves(xw2, jnp.ones((xw2.shape[0], 1), jnp.float32))


def _tc_tail_body(sp_ref, b2_ref, g2_ref, be2_ref, z1_ref, zb_ref,
                  wg1_ref, bg1_ref, wg2_ref, bg2_ref, wc_ref, bc_ref, out_ref):
    q, d = _unpack_sums(sp_ref[...])
    dinv = jnp.where(d > 0, 1.0 / jnp.where(d > 0, d, 1.0), 0.0)
    h = dinv * q + b2_ref[...]
    z2 = _ln_leaky(h, g2_ref[...], be2_ref[...])
    zsp = z1_ref[...] + z2
    zb = zb_ref[...]
    comb = jnp.concatenate([zsp, zb], axis=1)
    se = jax.nn.relu(jnp.dot(comb, wg1_ref[...], preferred_element_type=jnp.float32)
                     + bg1_ref[...])
    gate = jax.nn.sigmoid(jnp.dot(se, wg2_ref[...], preferred_element_type=jnp.float32)
                          + bg2_ref[...])
    fused = gate * zsp + (1.0 - gate) * zb
    out_ref[...] = jnp.dot(fused, wc_ref[...], preferred_element_type=jnp.float32) + bc_ref[...]


def _row_spec(width):
    return pl.BlockSpec((R, width), lambda i: (i, 0))


def _tbl_spec():
    return pl.BlockSpec((2, R, FH), lambda i: (0, i, 0))


def _part_spec():
    return pl.BlockSpec((NC, 2, R, FH), lambda i: (0, 0, i, 0))


def _full_spec(shape):
    nd = len(shape)
    return pl.BlockSpec(shape, lambda i: (0,) * nd)


def kernel(x, edge_index, edge_weight, W1, b1, g1, be1, W2, b2, g2, be2,
           Wb, bb, gb, beb, Wg1, bg1, Wg2, bg2, Wc, bc):
    node = edge_index[0]
    he = edge_index[1]
    padlen = NPAD - NINC
    node_g = jnp.pad(node, (0, padlen)).reshape(NW * NCHW, CH)
    he_g = jnp.pad(he, (0, padlen)).reshape(NW * NCHW, CH)
    node_s = jnp.pad(node, (0, padlen), constant_values=PAD).reshape(NW * NCHW, CH)
    he_s = jnp.pad(he, (0, padlen), constant_values=PAD).reshape(NW * NCHW, CH)
    zblk = jnp.zeros((CH, FH), jnp.float32)
    hw_col = edge_weight.reshape(N, 1)

    grid = (N // R,)
    t1, zb = pl.pallas_call(
        _tc_pre_body,
        grid=grid,
        in_specs=[_row_spec(F), _full_spec((F, F)), _full_spec((F, F)),
                  _full_spec((F,)), _full_spec((F,)), _full_spec((F,))],
        out_specs=[_tbl_spec(), _row_spec(F)],
        out_shape=[jax.ShapeDtypeStruct((2, N, FH), jnp.float32),
                   jax.ShapeDtypeStruct((N, F), jnp.float32)],
    )(x, W1, Wb, bb, gb, beb)

    def sc_pass(tbl, gidx, sidx):
        tbl = jnp.pad(tbl, ((0, 0), (0, ACC_ROWS - N), (0, 0)))
        return _sc_seg_sum(tbl, gidx, sidx, zblk)

    def tc_scale(spart):
        return pl.pallas_call(
            _tc_scale_body,
            grid=grid,
            in_specs=[_part_spec(), _row_spec(1)],
            out_specs=_tbl_spec(),
            out_shape=jax.ShapeDtypeStruct((2, N, FH), jnp.float32),
        )(spart, hw_col)

    s1 = sc_pass(t1, node_g, he_s)
    t2 = tc_scale(s1)
    s2 = sc_pass(t2, he_g, node_s)
    z1, t3 = pl.pallas_call(
        _tc_z1_body,
        grid=grid,
        in_specs=[_part_spec(), _full_spec((F,)), _full_spec((F,)),
                  _full_spec((F,)), _full_spec((F, F))],
        out_specs=[_row_spec(F), _tbl_spec()],
        out_shape=[jax.ShapeDtypeStruct((N, F), jnp.float32),
                   jax.ShapeDtypeStruct((2, N, FH), jnp.float32)],
    )(s2, b1, g1, be1, W2)

    s3 = sc_pass(t3, node_g, he_s)
    t4 = tc_scale(s3)
    s4 = sc_pass(t4, he_g, node_s)

    out = pl.pallas_call(
        _tc_tail_body,
        grid=grid,
        in_specs=[_part_spec(), _full_spec((F,)), _full_spec((F,)),
                  _full_spec((F,)), _row_spec(F), _row_spec(F),
                  _full_spec((2 * F, F // 2)), _full_spec((F // 2,)),
                  _full_spec((F // 2, F)), _full_spec((F,)),
                  _full_spec((F, 1)), _full_spec((1,))],
        out_specs=pl.BlockSpec((R, 1), lambda i: (i, 0)),
        out_shape=jax.ShapeDtypeStruct((N, 1), jnp.float32),
    )(s4, b2, g2, be2, z1, zb, Wg1, bg1, Wg2, bg2, Wc, bc)
    return out


# speedup vs baseline: 1.9674x; 1.0282x over previous
import functools

import jax
import jax.numpy as jnp
from jax import lax
from jax.experimental import pallas as pl
from jax.experimental.pallas import tpu as pltpu
from jax.experimental.pallas import tpu_sc as plsc

N = 10000
F = 128
FH = 80
HF = F // 2
NINC = 320000
NC = 2
NS = 16
NW = NC * NS
CH = 128
RND = 40
NRND = 2
NCHW = RND * NRND
NPAD = NW * NCHW * CH
ACC_ROWS = 10016
TROWS = ACC_ROWS // NS
PAD = N

_mesh = plsc.VectorSubcoreMesh(core_axis_name="c", subcore_axis_name="s",
                               num_cores=NC, num_subcores=NS)


@functools.partial(
    pl.kernel,
    out_type=jax.ShapeDtypeStruct((NC, 2, ACC_ROWS, FH), jnp.float32),
    mesh=_mesh,
    scratch_types=[
        pltpu.VMEM((RND, CH), jnp.int32),
        pltpu.VMEM((RND, CH), jnp.int32),
        pltpu.VMEM((CH, FH), jnp.float32),
        pltpu.VMEM((CH, FH), jnp.float32),
        pltpu.VMEM_SHARED((ACC_ROWS, FH), jnp.float32),
        pltpu.VMEM_SHARED((ACC_ROWS, FH), jnp.float32),
        pltpu.SemaphoreType.DMA,
        pltpu.SemaphoreType.DMA,
    ],
    compiler_params=pltpu.CompilerParams(use_tc_tiling_on_sc=False),
)
def _sc_seg_sum(tbl, gidx, sidx, zblk, out, gi, si, rows_a, rows_b,
                tsp, acc, sem_a, sem_b):
    c = lax.axis_index("c")
    s = lax.axis_index("s")
    w = c * NS + s
    r0 = s * TROWS

    for h in range(2):
        st = pltpu.async_copy(tbl.at[h, pl.ds(r0, TROWS)],
                              tsp.at[pl.ds(r0, TROWS)], sem_a)
        zr = pltpu.async_copy(zblk, acc.at[pl.ds(r0, TROWS)], sem_b)
        st.wait()
        zr.wait()
        plsc.subcore_barrier()

        def round_body(r, carry):
            row0 = (w * NRND + r) * RND
            pltpu.sync_copy(gidx.at[pl.ds(row0, RND)], gi)
            pltpu.sync_copy(sidx.at[pl.ds(row0, RND)], si)
            pltpu.async_copy(tsp.at[gi.at[0]], rows_a, sem_a)

            def body(j, carry2):
                c0 = 2 * j
                pltpu.async_copy(tsp.at[gi.at[c0 + 1]], rows_b, sem_b)
                pltpu.make_async_copy(tsp.at[gi.at[c0]], rows_a, sem_a).wait()
                pltpu.sync_copy(rows_a, acc.at[si.at[c0]], add=True)
                pltpu.async_copy(tsp.at[gi.at[c0 + 2]], rows_a, sem_a)
                pltpu.make_async_copy(tsp.at[gi.at[c0 + 1]], rows_b, sem_b).wait()
                pltpu.sync_copy(rows_b, acc.at[si.at[c0 + 1]], add=True)
                return carry2

            lax.fori_loop(0, RND // 2 - 1, body, 0)
            pltpu.async_copy(tsp.at[gi.at[RND - 1]], rows_b, sem_b)
            pltpu.make_async_copy(tsp.at[gi.at[RND - 2]], rows_a, sem_a).wait()
            pltpu.sync_copy(rows_a, acc.at[si.at[RND - 2]], add=True)
            pltpu.make_async_copy(tsp.at[gi.at[RND - 1]], rows_b, sem_b).wait()
            pltpu.sync_copy(rows_b, acc.at[si.at[RND - 1]], add=True)
            return carry

        lax.fori_loop(0, NRND, round_body, 0)
        plsc.subcore_barrier()
        pltpu.sync_copy(acc.at[pl.ds(r0, TROWS)], out.at[c, h, pl.ds(r0, TROWS)])


R = 1000


def _ln_leaky(h, g, b):
    mu = jnp.mean(h, axis=-1, keepdims=True)
    var = jnp.mean((h - mu) ** 2, axis=-1, keepdims=True)
    h = (h - mu) / jnp.sqrt(var + 1e-5) * g + b
    return jnp.where(h >= 0, h, 0.2 * h)


def _pack_halves(feats, extra):
    z = jnp.zeros((feats.shape[0], FH - HF - 1), jnp.float32)
    ta = jnp.concatenate([feats[:, :HF], extra, z], axis=1)
    tb = jnp.concatenate([feats[:, HF:], jnp.zeros_like(extra), z], axis=1)
    return jnp.stack([ta, tb])


def _unpack_sums(sp):
    qa = sp[0, 0] + sp[1, 0]
    qb = sp[0, 1] + sp[1, 1]
    return jnp.concatenate([qa[:, :HF], qb[:, :HF]], axis=1), qa[:, HF:HF + 1]


def _tc_pre_body(x_ref, w1_ref, wb_ref, bb_ref, gb_ref, beb_ref,
                 t1_ref, zb_ref):
    xb = x_ref[...]
    xw1 = jnp.dot(xb, w1_ref[...], preferred_element_type=jnp.float32)
    t1_ref[...] = _pack_halves(xw1, jnp.ones((xw1.shape[0], 1), jnp.float32))
    h = jnp.dot(xb, wb_ref[...], preferred_element_type=jnp.float32) + bb_ref[...]
    zb_ref[...] = _ln_leaky(h, gb_ref[...], beb_ref[...])


def _tc_scale_body(sp_ref, hw_ref, t_ref):
    q, bcnt = _unpack_sums(sp_ref[...])
    binv = jnp.where(bcnt > 0, 1.0 / jnp.where(bcnt > 0, bcnt, 1.0), 0.0)
    t_ref[...] = _pack_halves(binv * q, hw_ref[...])


def _tc_z1_body(sp_ref, b1_ref, g1_ref, be1_ref, w2_ref, z1_ref, t3_ref):
    q, d = _unpack_sums(sp_ref[...])
    dinv = jnp.where(d > 0, 1.0 / jnp.where(d > 0, d, 1.0), 0.0)
    h = dinv * q + b1_ref[...]
    z1 = _ln_leaky(h, g1_ref[...], be1_ref[...])
    z1_ref[...] = z1
    xw2 = jnp.dot(z1, w2_ref[...], preferred_element_type=jnp.float32)
    t3_ref[...] = _pack_halves(xw2, jnp.ones((xw2.shape[0], 1), jnp.float32))


def _tc_tail_body(sp_ref, b2_ref, g2_ref, be2_ref, z1_ref, zb_ref,
                  wg1_ref, bg1_ref, wg2_ref, bg2_ref, wc_ref, bc_ref, out_ref):
    q, d = _unpack_sums(sp_ref[...])
    dinv = jnp.where(d > 0, 1.0 / jnp.where(d > 0, d, 1.0), 0.0)
    h = dinv * q + b2_ref[...]
    z2 = _ln_leaky(h, g2_ref[...], be2_ref[...])
    zsp = z1_ref[...] + z2
    zb = zb_ref[...]
    comb = jnp.concatenate([zsp, zb], axis=1)
    se = jax.nn.relu(jnp.dot(comb, wg1_ref[...], preferred_element_type=jnp.float32)
                     + bg1_ref[...])
    gate = jax.nn.sigmoid(jnp.dot(se, wg2_ref[...], preferred_element_type=jnp.float32)
                          + bg2_ref[...])
    fused = gate * zsp + (1.0 - gate) * zb
    out_ref[...] = jnp.dot(fused, wc_ref[...], preferred_element_type=jnp.float32) + bc_ref[...]


def _row_spec(width):
    return pl.BlockSpec((R, width), lambda i: (i, 0))


def _tbl_spec():
    return pl.BlockSpec((2, R, FH), lambda i: (0, i, 0))


def _part_spec():
    return pl.BlockSpec((NC, 2, R, FH), lambda i: (0, 0, i, 0))


def _full_spec(shape):
    nd = len(shape)
    return pl.BlockSpec(shape, lambda i: (0,) * nd)


def kernel(x, edge_index, edge_weight, W1, b1, g1, be1, W2, b2, g2, be2,
           Wb, bb, gb, beb, Wg1, bg1, Wg2, bg2, Wc, bc):
    node = edge_index[0]
    he = edge_index[1]
    padlen = NPAD - NINC
    node_g = jnp.pad(node, (0, padlen)).reshape(NW * NCHW, CH)
    he_g = jnp.pad(he, (0, padlen)).reshape(NW * NCHW, CH)
    node_s = jnp.pad(node, (0, padlen), constant_values=PAD).reshape(NW * NCHW, CH)
    he_s = jnp.pad(he, (0, padlen), constant_values=PAD).reshape(NW * NCHW, CH)
    zblk = jnp.zeros((TROWS, FH), jnp.float32)
    hw_col = edge_weight.reshape(N, 1)

    grid = (N // R,)
    t1, zb = pl.pallas_call(
        _tc_pre_body,
        grid=grid,
        in_specs=[_row_spec(F), _full_spec((F, F)), _full_spec((F, F)),
                  _full_spec((F,)), _full_spec((F,)), _full_spec((F,))],
        out_specs=[_tbl_spec(), _row_spec(F)],
        out_shape=[jax.ShapeDtypeStruct((2, N, FH), jnp.float32),
                   jax.ShapeDtypeStruct((N, F), jnp.float32)],
    )(x, W1, Wb, bb, gb, beb)

    def sc_pass(tbl, gidx, sidx):
        tbl = jnp.pad(tbl, ((0, 0), (0, ACC_ROWS - N), (0, 0)))
        return _sc_seg_sum(tbl, gidx, sidx, zblk)

    def tc_scale(spart):
        return pl.pallas_call(
            _tc_scale_body,
            grid=grid,
            in_specs=[_part_spec(), _row_spec(1)],
            out_specs=_tbl_spec(),
            out_shape=jax.ShapeDtypeStruct((2, N, FH), jnp.float32),
        )(spart, hw_col)

    s1 = sc_pass(t1, node_g, he_s)
    t2 = tc_scale(s1)
    s2 = sc_pass(t2, he_g, node_s)
    z1, t3 = pl.pallas_call(
        _tc_z1_body,
        grid=grid,
        in_specs=[_part_spec(), _full_spec((F,)), _full_spec((F,)),
                  _full_spec((F,)), _full_spec((F, F))],
        out_specs=[_row_spec(F), _tbl_spec()],
        out_shape=[jax.ShapeDtypeStruct((N, F), jnp.float32),
                   jax.ShapeDtypeStruct((2, N, FH), jnp.float32)],
    )(s2, b1, g1, be1, W2)

    s3 = sc_pass(t3, node_g, he_s)
    t4 = tc_scale(s3)
    s4 = sc_pass(t4, he_g, node_s)

    out = pl.pallas_call(
        _tc_tail_body,
        grid=grid,
        in_specs=[_part_spec(), _full_spec((F,)), _full_spec((F,)),
                  _full_spec((F,)), _row_spec(F), _row_spec(F),
                  _full_spec((2 * F, F // 2)), _full_spec((F // 2,)),
                  _full_spec((F // 2, F)), _full_spec((F,)),
                  _full_spec((F, 1)), _full_spec((1,))],
        out_specs=pl.BlockSpec((R, 1), lambda i: (i, 0)),
        out_shape=jax.ShapeDtypeStruct((N, 1), jnp.float32),
    )(s4, b2, g2, be2, z1, zb, Wg1, bg1, Wg2, bg2, Wc, bc)
    return out


# speedup vs baseline: 2.0219x; 1.0277x over previous
import functools

import jax
import jax.numpy as jnp
from jax import lax
from jax.experimental import pallas as pl
from jax.experimental.pallas import tpu as pltpu
from jax.experimental.pallas import tpu_sc as plsc

N = 10000
F = 128
FH = 80
HF = F // 2
NINC = 320000
NC = 2
NS = 16
NW = NC * NS
CH = 128
RND = 40
NRND = 2
NCHW = RND * NRND
NPAD = NW * NCHW * CH
ACC_ROWS = 10016
TROWS = ACC_ROWS // NS
PAD = N

_mesh = plsc.VectorSubcoreMesh(core_axis_name="c", subcore_axis_name="s",
                               num_cores=NC, num_subcores=NS)


@functools.partial(
    pl.kernel,
    out_type=jax.ShapeDtypeStruct((NC, 2, ACC_ROWS, FH), jnp.float32),
    mesh=_mesh,
    scratch_types=[
        pltpu.VMEM((RND, CH), jnp.int32),
        pltpu.VMEM((RND, CH), jnp.int32),
        pltpu.VMEM((CH, FH), jnp.float32),
        pltpu.VMEM((CH, FH), jnp.float32),
        pltpu.VMEM_SHARED((ACC_ROWS, FH), jnp.float32),
        pltpu.VMEM_SHARED((ACC_ROWS, FH), jnp.float32),
        pltpu.SemaphoreType.DMA,
        pltpu.SemaphoreType.DMA,
    ],
    compiler_params=pltpu.CompilerParams(use_tc_tiling_on_sc=False),
)
def _sc_seg_sum(tbl, gidx, sidx, zblk, out, gi, si, rows_a, rows_b,
                tsp, acc, sem_a, sem_b):
    c = lax.axis_index("c")
    s = lax.axis_index("s")
    w = c * NS + s
    r0 = s * TROWS

    for h in range(2):
        st = pltpu.async_copy(tbl.at[h, pl.ds(r0, TROWS)],
                              tsp.at[pl.ds(r0, TROWS)], sem_a)
        zr = pltpu.async_copy(zblk, acc.at[pl.ds(r0, TROWS)], sem_b)
        st.wait()
        zr.wait()
        plsc.subcore_barrier()

        def round_body(r, carry):
            row0 = (w * NRND + r) * RND
            pltpu.sync_copy(gidx.at[pl.ds(row0, RND)], gi)
            pltpu.sync_copy(sidx.at[pl.ds(row0, RND)], si)
            pltpu.async_copy(tsp.at[gi.at[0]], rows_a, sem_a)

            def body(j, carry2):
                c0 = 2 * j
                pltpu.async_copy(tsp.at[gi.at[c0 + 1]], rows_b, sem_b)
                pltpu.make_async_copy(tsp.at[gi.at[c0]], rows_a, sem_a).wait()
                pltpu.sync_copy(rows_a, acc.at[si.at[c0]], add=True)
                pltpu.async_copy(tsp.at[gi.at[c0 + 2]], rows_a, sem_a)
                pltpu.make_async_copy(tsp.at[gi.at[c0 + 1]], rows_b, sem_b).wait()
                pltpu.sync_copy(rows_b, acc.at[si.at[c0 + 1]], add=True)
                return carry2

            lax.fori_loop(0, RND // 2 - 1, body, 0)
            pltpu.async_copy(tsp.at[gi.at[RND - 1]], rows_b, sem_b)
            pltpu.make_async_copy(tsp.at[gi.at[RND - 2]], rows_a, sem_a).wait()
            pltpu.sync_copy(rows_a, acc.at[si.at[RND - 2]], add=True)
            pltpu.make_async_copy(tsp.at[gi.at[RND - 1]], rows_b, sem_b).wait()
            pltpu.sync_copy(rows_b, acc.at[si.at[RND - 1]], add=True)
            return carry

        lax.fori_loop(0, NRND, round_body, 0)
        plsc.subcore_barrier()
        pltpu.sync_copy(acc.at[pl.ds(r0, TROWS)], out.at[c, h, pl.ds(r0, TROWS)])


R = 1000


def _ln_leaky(h, g, b):
    mu = jnp.mean(h, axis=-1, keepdims=True)
    var = jnp.mean((h - mu) ** 2, axis=-1, keepdims=True)
    h = (h - mu) / jnp.sqrt(var + 1e-5) * g + b
    return jnp.where(h >= 0, h, 0.2 * h)


def _pack_halves(feats, extra):
    z = jnp.zeros((feats.shape[0], FH - HF - 1), jnp.float32)
    ta = jnp.concatenate([feats[:, :HF], extra, z], axis=1)
    tb = jnp.concatenate([feats[:, HF:], jnp.zeros_like(extra), z], axis=1)
    return jnp.stack([ta, tb])


def _unpack_sums(sp):
    qa = sp[0, 0] + sp[1, 0]
    qb = sp[0, 1] + sp[1, 1]
    return jnp.concatenate([qa[:, :HF], qb[:, :HF]], axis=1), qa[:, HF:HF + 1]


def _tc_pre_body(x_ref, w1_ref, wb_ref, bb_ref, gb_ref, beb_ref,
                 t1_ref, zb_ref):
    xb = x_ref[...]
    xw1 = jnp.dot(xb, w1_ref[...], preferred_element_type=jnp.float32)
    t1_ref[...] = _pack_halves(xw1, jnp.ones((xw1.shape[0], 1), jnp.float32))
    h = jnp.dot(xb, wb_ref[...], preferred_element_type=jnp.float32) + bb_ref[...]
    zb_ref[...] = _ln_leaky(h, gb_ref[...], beb_ref[...])


def _tc_scale_body(sp_ref, hw_ref, t_ref):
    q, bcnt = _unpack_sums(sp_ref[...])
    binv = jnp.where(bcnt > 0, 1.0 / jnp.where(bcnt > 0, bcnt, 1.0), 0.0)
    t_ref[...] = _pack_halves(binv * q, hw_ref[...])


def _tc_z1_body(sp_ref, b1_ref, g1_ref, be1_ref, w2_ref, z1_ref, t3_ref):
    q, d = _unpack_sums(sp_ref[...])
    dinv = jnp.where(d > 0, 1.0 / jnp.where(d > 0, d, 1.0), 0.0)
    h = dinv * q + b1_ref[...]
    z1 = _ln_leaky(h, g1_ref[...], be1_ref[...])
    z1_ref[...] = z1
    xw2 = jnp.dot(z1, w2_ref[...], preferred_element_type=jnp.float32)
    t3_ref[...] = _pack_halves(xw2, jnp.ones((xw2.shape[0], 1), jnp.float32))


def _tc_tail_body(sp_ref, b2_ref, g2_ref, be2_ref, z1_ref, zb_ref,
                  wg1_ref, bg1_ref, wg2_ref, bg2_ref, wc_ref, bc_ref, out_ref):
    q, d = _unpack_sums(sp_ref[...])
    dinv = jnp.where(d > 0, 1.0 / jnp.where(d > 0, d, 1.0), 0.0)
    h = dinv * q + b2_ref[...]
    z2 = _ln_leaky(h, g2_ref[...], be2_ref[...])
    zsp = z1_ref[...] + z2
    zb = zb_ref[...]
    comb = jnp.concatenate([zsp, zb], axis=1)
    se = jax.nn.relu(jnp.dot(comb, wg1_ref[...], preferred_element_type=jnp.float32)
                     + bg1_ref[...])
    gate = jax.nn.sigmoid(jnp.dot(se, wg2_ref[...], preferred_element_type=jnp.float32)
                          + bg2_ref[...])
    fused = gate * zsp + (1.0 - gate) * zb
    out_ref[...] = jnp.dot(fused, wc_ref[...], preferred_element_type=jnp.float32) + bc_ref[...]


def _row_spec(width):
    return pl.BlockSpec((R, width), lambda i: (i, 0))


def _tbl_spec():
    return pl.BlockSpec((2, R, FH), lambda i: (0, i, 0))


def _part_spec():
    return pl.BlockSpec((NC, 2, R, FH), lambda i: (0, 0, i, 0))


def _full_spec(shape):
    nd = len(shape)
    return pl.BlockSpec(shape, lambda i: (0,) * nd)


def kernel(x, edge_index, edge_weight, W1, b1, g1, be1, W2, b2, g2, be2,
           Wb, bb, gb, beb, Wg1, bg1, Wg2, bg2, Wc, bc):
    node = edge_index[0]
    he = edge_index[1]
    padlen = NPAD - NINC
    node_g = jnp.pad(node, (0, padlen)).reshape(NW * NCHW, CH)
    he_g = jnp.pad(he, (0, padlen)).reshape(NW * NCHW, CH)
    node_s = jnp.pad(node, (0, padlen), constant_values=PAD).reshape(NW * NCHW, CH)
    he_s = jnp.pad(he, (0, padlen), constant_values=PAD).reshape(NW * NCHW, CH)
    zblk = jnp.zeros((TROWS, FH), jnp.float32)
    hw_col = edge_weight.reshape(N, 1)

    grid = (N // R,)
    t1, zb = pl.pallas_call(
        _tc_pre_body,
        grid=grid,
        in_specs=[_row_spec(F), _full_spec((F, F)), _full_spec((F, F)),
                  _full_spec((F,)), _full_spec((F,)), _full_spec((F,))],
        out_specs=[_tbl_spec(), _row_spec(F)],
        out_shape=[jax.ShapeDtypeStruct((2, ACC_ROWS, FH), jnp.float32),
                   jax.ShapeDtypeStruct((N, F), jnp.float32)],
    )(x, W1, Wb, bb, gb, beb)

    def sc_pass(tbl, gidx, sidx):
        return _sc_seg_sum(tbl, gidx, sidx, zblk)

    def tc_scale(spart):
        return pl.pallas_call(
            _tc_scale_body,
            grid=grid,
            in_specs=[_part_spec(), _row_spec(1)],
            out_specs=_tbl_spec(),
            out_shape=jax.ShapeDtypeStruct((2, ACC_ROWS, FH), jnp.float32),
        )(spart, hw_col)

    s1 = sc_pass(t1, node_g, he_s)
    t2 = tc_scale(s1)
    s2 = sc_pass(t2, he_g, node_s)
    z1, t3 = pl.pallas_call(
        _tc_z1_body,
        grid=grid,
        in_specs=[_part_spec(), _full_spec((F,)), _full_spec((F,)),
                  _full_spec((F,)), _full_spec((F, F))],
        out_specs=[_row_spec(F), _tbl_spec()],
        out_shape=[jax.ShapeDtypeStruct((N, F), jnp.float32),
                   jax.ShapeDtypeStruct((2, ACC_ROWS, FH), jnp.float32)],
    )(s2, b1, g1, be1, W2)

    s3 = sc_pass(t3, node_g, he_s)
    t4 = tc_scale(s3)
    s4 = sc_pass(t4, he_g, node_s)

    out = pl.pallas_call(
        _tc_tail_body,
        grid=grid,
        in_specs=[_part_spec(), _full_spec((F,)), _full_spec((F,)),
                  _full_spec((F,)), _row_spec(F), _row_spec(F),
                  _full_spec((2 * F, F // 2)), _full_spec((F // 2,)),
                  _full_spec((F // 2, F)), _full_spec((F,)),
                  _full_spec((F, 1)), _full_spec((1,))],
        out_specs=pl.BlockSpec((R, 1), lambda i: (i, 0)),
        out_shape=jax.ShapeDtypeStruct((N, 1), jnp.float32),
    )(s4, b2, g2, be2, z1, zb, Wg1, bg1, Wg2, bg2, Wc, bc)
    return out


# speedup vs baseline: 2.0273x; 1.0027x over previous
import functools

import jax
import jax.numpy as jnp
from jax import lax
from jax.experimental import pallas as pl
from jax.experimental.pallas import tpu as pltpu
from jax.experimental.pallas import tpu_sc as plsc

N = 10000
F = 128
FH = 80
HF = F // 2
NINC = 320000
NC = 2
NS = 16
NW = NC * NS
CH = 128
RND = 40
NRND = 2
NCHW = RND * NRND
NPAD = NW * NCHW * CH
ACC_ROWS = 10016
TROWS = ACC_ROWS // NS
PAD = N

_mesh = plsc.VectorSubcoreMesh(core_axis_name="c", subcore_axis_name="s",
                               num_cores=NC, num_subcores=NS)


@functools.partial(
    pl.kernel,
    out_type=jax.ShapeDtypeStruct((NC, 2, ACC_ROWS, FH), jnp.float32),
    mesh=_mesh,
    scratch_types=[
        pltpu.VMEM((RND, CH), jnp.int32),
        pltpu.VMEM((RND, CH), jnp.int32),
        pltpu.VMEM((CH, FH), jnp.float32),
        pltpu.VMEM((CH, FH), jnp.float32),
        pltpu.VMEM_SHARED((ACC_ROWS, FH), jnp.float32),
        pltpu.VMEM_SHARED((ACC_ROWS, FH), jnp.float32),
        pltpu.SemaphoreType.DMA,
        pltpu.SemaphoreType.DMA,
    ],
    compiler_params=pltpu.CompilerParams(use_tc_tiling_on_sc=False),
)
def _sc_seg_sum(tbl, gidx, sidx, zblk, out, gi, si, rows_a, rows_b,
                tsp, acc, sem_a, sem_b):
    c = lax.axis_index("c")
    s = lax.axis_index("s")
    w = c * NS + s
    r0 = s * TROWS

    st = pltpu.async_copy(tbl.at[0, pl.ds(r0, TROWS)],
                          tsp.at[pl.ds(r0, TROWS)], sem_a)
    zr = pltpu.async_copy(zblk, acc.at[pl.ds(r0, TROWS)], sem_b)
    st.wait()
    zr.wait()
    plsc.subcore_barrier()

    for h in range(2):
        def round_body(r, carry):
            row0 = (w * NRND + r) * RND
            pltpu.sync_copy(gidx.at[pl.ds(row0, RND)], gi)
            pltpu.sync_copy(sidx.at[pl.ds(row0, RND)], si)
            pltpu.async_copy(tsp.at[gi.at[0]], rows_a, sem_a)

            def body(j, carry2):
                c0 = 2 * j
                pltpu.async_copy(tsp.at[gi.at[c0 + 1]], rows_b, sem_b)
                pltpu.make_async_copy(tsp.at[gi.at[c0]], rows_a, sem_a).wait()
                pltpu.sync_copy(rows_a, acc.at[si.at[c0]], add=True)
                pltpu.async_copy(tsp.at[gi.at[c0 + 2]], rows_a, sem_a)
                pltpu.make_async_copy(tsp.at[gi.at[c0 + 1]], rows_b, sem_b).wait()
                pltpu.sync_copy(rows_b, acc.at[si.at[c0 + 1]], add=True)
                return carry2

            lax.fori_loop(0, RND // 2 - 1, body, 0)
            pltpu.async_copy(tsp.at[gi.at[RND - 1]], rows_b, sem_b)
            pltpu.make_async_copy(tsp.at[gi.at[RND - 2]], rows_a, sem_a).wait()
            pltpu.sync_copy(rows_a, acc.at[si.at[RND - 2]], add=True)
            pltpu.make_async_copy(tsp.at[gi.at[RND - 1]], rows_b, sem_b).wait()
            pltpu.sync_copy(rows_b, acc.at[si.at[RND - 1]], add=True)
            return carry

        lax.fori_loop(0, NRND, round_body, 0)
        plsc.subcore_barrier()
        co = pltpu.async_copy(acc.at[pl.ds(r0, TROWS)],
                              out.at[c, h, pl.ds(r0, TROWS)], sem_a)
        if h == 0:
            st = pltpu.async_copy(tbl.at[1, pl.ds(r0, TROWS)],
                                  tsp.at[pl.ds(r0, TROWS)], sem_b)
        co.wait()
        if h == 0:
            zr = pltpu.async_copy(zblk, acc.at[pl.ds(r0, TROWS)], sem_a)
            st.wait()
            zr.wait()
            plsc.subcore_barrier()


R = 1000


def _ln_leaky(h, g, b):
    mu = jnp.mean(h, axis=-1, keepdims=True)
    var = jnp.mean((h - mu) ** 2, axis=-1, keepdims=True)
    h = (h - mu) / jnp.sqrt(var + 1e-5) * g + b
    return jnp.where(h >= 0, h, 0.2 * h)


def _pack_halves(feats, extra):
    z = jnp.zeros((feats.shape[0], FH - HF - 1), jnp.float32)
    ta = jnp.concatenate([feats[:, :HF], extra, z], axis=1)
    tb = jnp.concatenate([feats[:, HF:], jnp.zeros_like(extra), z], axis=1)
    return jnp.stack([ta, tb])


def _unpack_sums(sp):
    qa = sp[0, 0] + sp[1, 0]
    qb = sp[0, 1] + sp[1, 1]
    return jnp.concatenate([qa[:, :HF], qb[:, :HF]], axis=1), qa[:, HF:HF + 1]


def _tc_pre_body(x_ref, w1_ref, wb_ref, bb_ref, gb_ref, beb_ref,
                 t1_ref, zb_ref):
    xb = x_ref[...]
    xw1 = jnp.dot(xb, w1_ref[...], preferred_element_type=jnp.float32)
    t1_ref[...] = _pack_halves(xw1, jnp.ones((xw1.shape[0], 1), jnp.float32))
    h = jnp.dot(xb, wb_ref[...], preferred_element_type=jnp.float32) + bb_ref[...]
    zb_ref[...] = _ln_leaky(h, gb_ref[...], beb_ref[...])


def _tc_scale_body(sp_ref, hw_ref, t_ref):
    q, bcnt = _unpack_sums(sp_ref[...])
    binv = jnp.where(bcnt > 0, 1.0 / jnp.where(bcnt > 0, bcnt, 1.0), 0.0)
    t_ref[...] = _pack_halves(binv * q, hw_ref[...])


def _tc_z1_body(sp_ref, b1_ref, g1_ref, be1_ref, w2_ref, z1_ref, t3_ref):
    q, d = _unpack_sums(sp_ref[...])
    dinv = jnp.where(d > 0, 1.0 / jnp.where(d > 0, d, 1.0), 0.0)
    h = dinv * q + b1_ref[...]
    z1 = _ln_leaky(h, g1_ref[...], be1_ref[...])
    z1_ref[...] = z1
    xw2 = jnp.dot(z1, w2_ref[...], preferred_element_type=jnp.float32)
    t3_ref[...] = _pack_halves(xw2, jnp.ones((xw2.shape[0], 1), jnp.float32))


def _tc_tail_body(sp_ref, b2_ref, g2_ref, be2_ref, z1_ref, zb_ref,
                  wg1_ref, bg1_ref, wg2_ref, bg2_ref, wc_ref, bc_ref, out_ref):
    q, d = _unpack_sums(sp_ref[...])
    dinv = jnp.where(d > 0, 1.0 / jnp.where(d > 0, d, 1.0), 0.0)
    h = dinv * q + b2_ref[...]
    z2 = _ln_leaky(h, g2_ref[...], be2_ref[...])
    zsp = z1_ref[...] + z2
    zb = zb_ref[...]
    comb = jnp.concatenate([zsp, zb], axis=1)
    se = jax.nn.relu(jnp.dot(comb, wg1_ref[...], preferred_element_type=jnp.float32)
                     + bg1_ref[...])
    gate = jax.nn.sigmoid(jnp.dot(se, wg2_ref[...], preferred_element_type=jnp.float32)
                          + bg2_ref[...])
    fused = gate * zsp + (1.0 - gate) * zb
    out_ref[...] = jnp.dot(fused, wc_ref[...], preferred_element_type=jnp.float32) + bc_ref[...]


def _row_spec(width):
    return pl.BlockSpec((R, width), lambda i: (i, 0))


def _tbl_spec():
    return pl.BlockSpec((2, R, FH), lambda i: (0, i, 0))


def _part_spec():
    return pl.BlockSpec((NC, 2, R, FH), lambda i: (0, 0, i, 0))


def _full_spec(shape):
    nd = len(shape)
    return pl.BlockSpec(shape, lambda i: (0,) * nd)


def kernel(x, edge_index, edge_weight, W1, b1, g1, be1, W2, b2, g2, be2,
           Wb, bb, gb, beb, Wg1, bg1, Wg2, bg2, Wc, bc):
    node = edge_index[0]
    he = edge_index[1]
    padlen = NPAD - NINC
    node_g = jnp.pad(node, (0, padlen)).reshape(NW * NCHW, CH)
    he_g = jnp.pad(he, (0, padlen)).reshape(NW * NCHW, CH)
    node_s = jnp.pad(node, (0, padlen), constant_values=PAD).reshape(NW * NCHW, CH)
    he_s = jnp.pad(he, (0, padlen), constant_values=PAD).reshape(NW * NCHW, CH)
    zblk = jnp.zeros((TROWS, FH), jnp.float32)
    hw_col = edge_weight.reshape(N, 1)

    grid = (N // R,)
    t1, zb = pl.pallas_call(
        _tc_pre_body,
        grid=grid,
        in_specs=[_row_spec(F), _full_spec((F, F)), _full_spec((F, F)),
                  _full_spec((F,)), _full_spec((F,)), _full_spec((F,))],
        out_specs=[_tbl_spec(), _row_spec(F)],
        out_shape=[jax.ShapeDtypeStruct((2, ACC_ROWS, FH), jnp.float32),
                   jax.ShapeDtypeStruct((N, F), jnp.float32)],
    )(x, W1, Wb, bb, gb, beb)

    def sc_pass(tbl, gidx, sidx):
        return _sc_seg_sum(tbl, gidx, sidx, zblk)

    def tc_scale(spart):
        return pl.pallas_call(
            _tc_scale_body,
            grid=grid,
            in_specs=[_part_spec(), _row_spec(1)],
            out_specs=_tbl_spec(),
            out_shape=jax.ShapeDtypeStruct((2, ACC_ROWS, FH), jnp.float32),
        )(spart, hw_col)

    s1 = sc_pass(t1, node_g, he_s)
    t2 = tc_scale(s1)
    s2 = sc_pass(t2, he_g, node_s)
    z1, t3 = pl.pallas_call(
        _tc_z1_body,
        grid=grid,
        in_specs=[_part_spec(), _full_spec((F,)), _full_spec((F,)),
                  _full_spec((F,)), _full_spec((F, F))],
        out_specs=[_row_spec(F), _tbl_spec()],
        out_shape=[jax.ShapeDtypeStruct((N, F), jnp.float32),
                   jax.ShapeDtypeStruct((2, ACC_ROWS, FH), jnp.float32)],
    )(s2, b1, g1, be1, W2)

    s3 = sc_pass(t3, node_g, he_s)
    t4 = tc_scale(s3)
    s4 = sc_pass(t4, he_g, node_s)

    out = pl.pallas_call(
        _tc_tail_body,
        grid=grid,
        in_specs=[_part_spec(), _full_spec((F,)), _full_spec((F,)),
                  _full_spec((F,)), _row_spec(F), _row_spec(F),
                  _full_spec((2 * F, F // 2)), _full_spec((F // 2,)),
                  _full_spec((F // 2, F)), _full_spec((F,)),
                  _full_spec((F, 1)), _full_spec((1,))],
        out_specs=pl.BlockSpec((R, 1), lambda i: (i, 0)),
        out_shape=jax.ShapeDtypeStruct((N, 1), jnp.float32),
    )(s4, b2, g2, be2, z1, zb, Wg1, bg1, Wg2, bg2, Wc, bc)
    return out
